# baseline (speedup 1.0000x reference)
import functools

import jax
import jax.numpy as jnp
from jax import lax
from jax.experimental import pallas as pl
from jax.experimental.pallas import tpu as pltpu
from jax.experimental.pallas import tpu_sc as plsc

N = 10000
NP = 10240
E = 640000
NCLASS = 16
HID = 128

NC = 2
NS = 16
NW = NC * NS

B = 80
E2 = 655360
EPT = E2 // NW
RPT = EPT // B
ROWS = E2 // B
NPT = NP // NS
DCH = 1280
SR = 16
NCHUNK = RPT // SR
RPT0 = 192
RPT1 = 320

_mesh = plsc.VectorSubcoreMesh(core_axis_name="c", subcore_axis_name="s")
_sc_params = pltpu.CompilerParams(needs_layout_passes=False)


@functools.partial(
    pl.kernel,
    out_type=jax.ShapeDtypeStruct((NW * NP,), jnp.float32),
    mesh=_mesh,
    compiler_params=_sc_params,
    scratch_types=[
        pltpu.VMEM((DCH,), jnp.int32),
        pltpu.VMEM((DCH,), jnp.float32),
        pltpu.VMEM((NP,), jnp.float32),
        pltpu.SemaphoreType.DMA,
    ],
)
def _deg_kernel(dst_hbm, ew_hbm, out_hbm, dst_v, ew_v, deg_v, sem):
    cid = lax.axis_index("c")
    sid = lax.axis_index("s")
    w = cid * NS + sid
    base = w * EPT
    zero = jnp.zeros((16,), jnp.float32)

    def zbody(i, _):
        deg_v[pl.ds(i * 16, 16)] = zero
        return 0

    lax.fori_loop(0, NP // 16, zbody, 0)

    def body(i, _):
        d = dst_v[pl.ds(i * 16, 16)]
        e = ew_v[pl.ds(i * 16, 16)]
        plsc.addupdate_scatter(deg_v, [d], e)
        return 0

    for c in range(EPT // DCH):
        cp1 = pltpu.async_copy(dst_hbm.at[pl.ds(base + c * DCH, DCH)], dst_v, sem)
        cp2 = pltpu.async_copy(ew_hbm.at[pl.ds(base + c * DCH, DCH)], ew_v, sem)
        cp1.wait()
        cp2.wait()
        lax.fori_loop(0, DCH // 16, body, 0)
    pltpu.sync_copy(deg_v, out_hbm.at[pl.ds(w * NP, NP)])


@functools.partial(
    pl.kernel,
    out_type=jax.ShapeDtypeStruct((NC, NP, HID), jnp.float32),
    mesh=_mesh,
    compiler_params=_sc_params,
    scratch_types=[
        pltpu.VMEM((SR, B), jnp.int32),
        pltpu.VMEM((SR, B), jnp.int32),
        pltpu.VMEM((SR, B), jnp.float32),
        pltpu.VMEM((B, HID), jnp.float32),
        pltpu.VMEM((B, HID), jnp.float32),
        pltpu.VMEM_SHARED((NP, HID), jnp.float32),
        pltpu.SemaphoreType.DMA,
        pltpu.SemaphoreType.DMA,
    ],
)
def _agg_kernel(src_hbm, dst_hbm, ew_hbm, h_hbm, acc_out,
                src_v, dst_v, ew_v, xb0, xb1, acc_sh,
                sem_st, sem_g):
    cid = lax.axis_index("c")
    sid = lax.axis_index("s")
    r0 = jnp.where(cid == 0, sid * RPT0, NS * RPT0 + sid * RPT1)
    nchunk = jnp.where(cid == 0, RPT0 // SR, RPT1 // SR)
    zero = jnp.zeros((16,), jnp.float32)
    gbufs = (xb0, xb1)

    def zbody(i, _):
        for k in range(HID // 16):
            xb0[i, pl.ds(k * 16, 16)] = zero
        return 0

    lax.fori_loop(0, B, zbody, 0)
    for q in range(NPT // B):
        pltpu.sync_copy(xb0, acc_sh.at[pl.ds(sid * NPT + q * B, B)])
    plsc.subcore_barrier()

    def gissue(r, buf):
        pltpu.async_copy(h_hbm.at[src_v.at[r]], buf, sem_g)

    def gwait(r, buf):
        pltpu.make_async_copy(h_hbm.at[src_v.at[r]], buf, sem_g).wait()

    def scale_row(r, buf):
        def ebody(jj, _):
            j = 2 * jj
            bc0 = plsc.load_gather(
                ew_v,
                [jnp.full((16,), r, jnp.int32), jnp.full((16,), j, jnp.int32)],
            )
            bc1 = plsc.load_gather(
                ew_v,
                [jnp.full((16,), r, jnp.int32), jnp.full((16,), j + 1, jnp.int32)],
            )
            for k in range(HID // 16):
                buf[j, pl.ds(k * 16, 16)] = buf[j, pl.ds(k * 16, 16)] * bc0
            for k in range(HID // 16):
                buf[j + 1, pl.ds(k * 16, 16)] = buf[j + 1, pl.ds(k * 16, 16)] * bc1
            return 0

        lax.fori_loop(0, B // 2, ebody, 0)

    def scatter_row(r, buf):
        pltpu.sync_copy(buf, acc_sh.at[dst_v.at[r]], add=True)

    def chunk_body(c, _):
        rc = r0 + c * SR
        cps = [
            pltpu.async_copy(src_hbm.at[pl.ds(rc, SR)], src_v, sem_st),
            pltpu.async_copy(dst_hbm.at[pl.ds(rc, SR)], dst_v, sem_st),
            pltpu.async_copy(ew_hbm.at[pl.ds(rc, SR)], ew_v, sem_st),
        ]
        for cp in cps:
            cp.wait()

        gissue(0, gbufs[0])

        def pair_body(p, _):
            r = 2 * p
            gissue(r + 1, gbufs[1])
            gwait(r, gbufs[0])
            scale_row(r, gbufs[0])
            scatter_row(r, gbufs[0])

            @pl.when(r + 2 < SR)
            def _():
                gissue(r + 2, gbufs[0])

            gwait(r + 1, gbufs[1])
            scale_row(r + 1, gbufs[1])
            scatter_row(r + 1, gbufs[1])
            return 0

        lax.fori_loop(0, SR // 2, pair_body, 0)
        return 0

    lax.fori_loop(0, nchunk, chunk_body, 0)
    plsc.subcore_barrier()
    pltpu.sync_copy(acc_sh.at[pl.ds(sid * NPT, NPT)],
                    acc_out.at[cid, pl.ds(sid * NPT, NPT)])


def _dinv_body(degp_ref, dinv_ref):
    s = jnp.sum(degp_ref[...], axis=0)
    dinv_ref[...] = lax.rsqrt(jnp.maximum(s, 1e-12))


def _g0_body(x_ref, dinv_ref, w_ref, o_ref):
    xs = x_ref[...] * dinv_ref[...]
    o_ref[...] = jnp.dot(xs, w_ref[...], preferred_element_type=jnp.float32)


def _h1_body(a0_ref, a1_ref, dinv_ref, b_ref, o_ref):
    dv = dinv_ref[...]
    agg = (a0_ref[...] + a1_ref[...]) * dv
    o_ref[...] = jnp.maximum(agg + b_ref[...], 0.0) * dv


def _out_body(a0_ref, a1_ref, dinv_ref, w_ref, b_ref, wfc_ref, bfc_ref, o_ref):
    agg = (a0_ref[...] + a1_ref[...]) * dinv_ref[...]
    h = jnp.maximum(
        jnp.dot(agg, w_ref[...], preferred_element_type=jnp.float32) + b_ref[...],
        0.0,
    )
    o_ref[...] = jnp.dot(h, wfc_ref[...], preferred_element_type=jnp.float32) + bfc_ref[...]


def kernel(x, edge_index, edge_weight, W1, b1, W2, b2, Wfc, bfc):
    src = edge_index[0].astype(jnp.int32)
    dst = edge_index[1].astype(jnp.int32)
    ew = edge_weight.astype(jnp.float32)

    pad = E2 - E - N
    loop = jnp.arange(N, dtype=jnp.int32)
    zpad = jnp.zeros((pad,), jnp.int32)
    src_e = jnp.concatenate([src, loop, zpad]).reshape(ROWS, B)
    dst_e = jnp.concatenate([dst, loop, zpad]).reshape(ROWS, B)
    ew_e = jnp.concatenate(
        [ew, jnp.ones((N,), jnp.float32), jnp.zeros((pad,), jnp.float32)]
    ).reshape(ROWS, B)

    deg_p = _deg_kernel(dst_e.reshape(E2), ew_e.reshape(E2))
    dinv = pl.pallas_call(
        _dinv_body,
        out_shape=jax.ShapeDtypeStruct((NP,), jnp.float32),
    )(deg_p.reshape(NW, NP))
    dinv_n = dinv[:N].reshape(N, 1)
    dinv_p = dinv.reshape(NP, 1)

    g0 = pl.pallas_call(
        _g0_body,
        out_shape=jax.ShapeDtypeStruct((N, HID), jnp.float32),
    )(x, dinv_n, W1)

    acc1 = _agg_kernel(src_e, dst_e, ew_e, g0)
    t1 = pl.pallas_call(
        _h1_body,
        out_shape=jax.ShapeDtypeStruct((NP, HID), jnp.float32),
    )(acc1[0], acc1[1], dinv_p, b1[None, :])

    acc2 = _agg_kernel(src_e, dst_e, ew_e, t1)
    out = pl.pallas_call(
        _out_body,
        out_shape=jax.ShapeDtypeStruct((NP, NCLASS), jnp.float32),
    )(acc2[0], acc2[1], dinv_p, W2, b2[None, :], Wfc, bfc[None, :])
    return out[:N]


# speedup vs baseline: 1.2538x; 1.2538x over previous
import functools

import jax
import jax.numpy as jnp
from jax import lax
from jax.experimental import pallas as pl
from jax.experimental.pallas import tpu as pltpu
from jax.experimental.pallas import tpu_sc as plsc

N = 10000
NP = 10240
E = 640000
NCLASS = 16
HID = 128

NC = 2
NS = 16
NW = NC * NS

B = 80
E2 = 655360
EPT = E2 // NW
RPT = EPT // B
ROWS = E2 // B
NPT = NP // NS
DCH = 1280
SR = 16
NCHUNK = RPT // SR
RPT0 = 320
RPT1 = 192

_mesh = plsc.VectorSubcoreMesh(core_axis_name="c", subcore_axis_name="s")
_sc_params = pltpu.CompilerParams(needs_layout_passes=False)


@functools.partial(
    pl.kernel,
    out_type=jax.ShapeDtypeStruct((NW * NP,), jnp.float32),
    mesh=_mesh,
    compiler_params=_sc_params,
    scratch_types=[
        pltpu.VMEM((DCH,), jnp.int32),
        pltpu.VMEM((DCH,), jnp.float32),
        pltpu.VMEM((NP,), jnp.float32),
        pltpu.SemaphoreType.DMA,
    ],
)
def _deg_kernel(dst_hbm, ew_hbm, out_hbm, dst_v, ew_v, deg_v, sem):
    cid = lax.axis_index("c")
    sid = lax.axis_index("s")
    w = cid * NS + sid
    base = w * EPT
    zero = jnp.zeros((16,), jnp.float32)

    def zbody(i, _):
        deg_v[pl.ds(i * 16, 16)] = zero
        return 0

    lax.fori_loop(0, NP // 16, zbody, 0)

    def body(i, _):
        d = dst_v[pl.ds(i * 16, 16)]
        e = ew_v[pl.ds(i * 16, 16)]
        plsc.addupdate_scatter(deg_v, [d], e)
        return 0

    for c in range(EPT // DCH):
        cp1 = pltpu.async_copy(dst_hbm.at[pl.ds(base + c * DCH, DCH)], dst_v, sem)
        cp2 = pltpu.async_copy(ew_hbm.at[pl.ds(base + c * DCH, DCH)], ew_v, sem)
        cp1.wait()
        cp2.wait()
        lax.fori_loop(0, DCH // 16, body, 0)
    pltpu.sync_copy(deg_v, out_hbm.at[pl.ds(w * NP, NP)])


@functools.partial(
    pl.kernel,
    out_type=jax.ShapeDtypeStruct((NC, NP, HID), jnp.float32),
    mesh=_mesh,
    compiler_params=_sc_params,
    scratch_types=[
        pltpu.VMEM((SR, B), jnp.int32),
        pltpu.VMEM((SR, B), jnp.int32),
        pltpu.VMEM((SR, B), jnp.float32),
        pltpu.VMEM((B, HID), jnp.float32),
        pltpu.VMEM((B, HID), jnp.float32),
        pltpu.VMEM_SHARED((NP, HID), jnp.float32),
        pltpu.SemaphoreType.DMA,
        pltpu.SemaphoreType.DMA,
    ],
)
def _agg_kernel(src_hbm, dst_hbm, ew_hbm, h_hbm, acc_out,
                src_v, dst_v, ew_v, xb0, xb1, acc_sh,
                sem_st, sem_g):
    cid = lax.axis_index("c")
    sid = lax.axis_index("s")
    r0 = jnp.where(cid == 0, sid * RPT0, NS * RPT0 + sid * RPT1)
    nchunk = jnp.where(cid == 0, RPT0 // SR, RPT1 // SR)
    zero = jnp.zeros((16,), jnp.float32)
    gbufs = (xb0, xb1)

    def zbody(i, _):
        for k in range(HID // 16):
            xb0[i, pl.ds(k * 16, 16)] = zero
        return 0

    lax.fori_loop(0, B, zbody, 0)
    for q in range(NPT // B):
        pltpu.sync_copy(xb0, acc_sh.at[pl.ds(sid * NPT + q * B, B)])
    plsc.subcore_barrier()

    def gissue(r, buf):
        pltpu.async_copy(h_hbm.at[src_v.at[r]], buf, sem_g)

    def gwait(r, buf):
        pltpu.make_async_copy(h_hbm.at[src_v.at[r]], buf, sem_g).wait()

    def scale_row(r, buf):
        def ebody(jj, _):
            j = 2 * jj
            bc0 = plsc.load_gather(
                ew_v,
                [jnp.full((16,), r, jnp.int32), jnp.full((16,), j, jnp.int32)],
            )
            bc1 = plsc.load_gather(
                ew_v,
                [jnp.full((16,), r, jnp.int32), jnp.full((16,), j + 1, jnp.int32)],
            )
            for k in range(HID // 16):
                buf[j, pl.ds(k * 16, 16)] = buf[j, pl.ds(k * 16, 16)] * bc0
            for k in range(HID // 16):
                buf[j + 1, pl.ds(k * 16, 16)] = buf[j + 1, pl.ds(k * 16, 16)] * bc1
            return 0

        lax.fori_loop(0, B // 2, ebody, 0)

    def scatter_row(r, buf):
        pltpu.sync_copy(buf, acc_sh.at[dst_v.at[r]], add=True)

    def chunk_body(c, _):
        rc = r0 + c * SR
        cps = [
            pltpu.async_copy(src_hbm.at[pl.ds(rc, SR)], src_v, sem_st),
            pltpu.async_copy(dst_hbm.at[pl.ds(rc, SR)], dst_v, sem_st),
            pltpu.async_copy(ew_hbm.at[pl.ds(rc, SR)], ew_v, sem_st),
        ]
        for cp in cps:
            cp.wait()

        gissue(0, gbufs[0])

        def pair_body(p, _):
            r = 2 * p
            gissue(r + 1, gbufs[1])
            gwait(r, gbufs[0])
            scale_row(r, gbufs[0])
            scatter_row(r, gbufs[0])

            @pl.when(r + 2 < SR)
            def _():
                gissue(r + 2, gbufs[0])

            gwait(r + 1, gbufs[1])
            scale_row(r + 1, gbufs[1])
            scatter_row(r + 1, gbufs[1])
            return 0

        lax.fori_loop(0, SR // 2, pair_body, 0)
        return 0

    lax.fori_loop(0, nchunk, chunk_body, 0)
    plsc.subcore_barrier()
    pltpu.sync_copy(acc_sh.at[pl.ds(sid * NPT, NPT)],
                    acc_out.at[cid, pl.ds(sid * NPT, NPT)])


def _dinv_body(degp_ref, dinv_ref):
    s = jnp.sum(degp_ref[...], axis=0)
    dinv_ref[...] = lax.rsqrt(jnp.maximum(s, 1e-12))


def _g0_body(x_ref, dinv_ref, w_ref, o_ref):
    xs = x_ref[...] * dinv_ref[...]
    o_ref[...] = jnp.dot(xs, w_ref[...], preferred_element_type=jnp.float32)


def _h1_body(a0_ref, a1_ref, dinv_ref, b_ref, o_ref):
    dv = dinv_ref[...]
    agg = (a0_ref[...] + a1_ref[...]) * dv
    o_ref[...] = jnp.maximum(agg + b_ref[...], 0.0) * dv


def _out_body(a0_ref, a1_ref, dinv_ref, w_ref, b_ref, wfc_ref, bfc_ref, o_ref):
    agg = (a0_ref[...] + a1_ref[...]) * dinv_ref[...]
    h = jnp.maximum(
        jnp.dot(agg, w_ref[...], preferred_element_type=jnp.float32) + b_ref[...],
        0.0,
    )
    o_ref[...] = jnp.dot(h, wfc_ref[...], preferred_element_type=jnp.float32) + bfc_ref[...]


def kernel(x, edge_index, edge_weight, W1, b1, W2, b2, Wfc, bfc):
    src = edge_index[0].astype(jnp.int32)
    dst = edge_index[1].astype(jnp.int32)
    ew = edge_weight.astype(jnp.float32)

    pad = E2 - E - N
    loop = jnp.arange(N, dtype=jnp.int32)
    zpad = jnp.zeros((pad,), jnp.int32)
    src_e = jnp.concatenate([src, loop, zpad]).reshape(ROWS, B)
    dst_e = jnp.concatenate([dst, loop, zpad]).reshape(ROWS, B)
    ew_e = jnp.concatenate(
        [ew, jnp.ones((N,), jnp.float32), jnp.zeros((pad,), jnp.float32)]
    ).reshape(ROWS, B)

    deg_p = _deg_kernel(dst_e.reshape(E2), ew_e.reshape(E2))
    dinv = pl.pallas_call(
        _dinv_body,
        out_shape=jax.ShapeDtypeStruct((NP,), jnp.float32),
    )(deg_p.reshape(NW, NP))
    dinv_n = dinv[:N].reshape(N, 1)
    dinv_p = dinv.reshape(NP, 1)

    g0 = pl.pallas_call(
        _g0_body,
        out_shape=jax.ShapeDtypeStruct((N, HID), jnp.float32),
    )(x, dinv_n, W1)

    acc1 = _agg_kernel(src_e, dst_e, ew_e, g0)
    t1 = pl.pallas_call(
        _h1_body,
        out_shape=jax.ShapeDtypeStruct((NP, HID), jnp.float32),
    )(acc1[0], acc1[1], dinv_p, b1[None, :])

    acc2 = _agg_kernel(src_e, dst_e, ew_e, t1)
    out = pl.pallas_call(
        _out_body,
        out_shape=jax.ShapeDtypeStruct((NP, NCLASS), jnp.float32),
    )(acc2[0], acc2[1], dinv_p, W2, b2[None, :], Wfc, bfc[None, :])
    return out[:N]
